# baseline (speedup 1.0000x reference)
import functools
import jax
import jax.numpy as jnp
from jax import lax
from jax.experimental import pallas as pl
from jax.experimental.pallas import tpu as pltpu
from jax.experimental.pallas import tpu_sc as plsc

N_NODES = 10000
D = 128
N_PAD = 10240
ACC_W = 144
E_RAW = 320000
E_TOT = E_RAW + N_NODES
N_WORKERS = 32
CHUNK = 10368
STAGE = 2592
G = 32
E_PAD = N_WORKERS * CHUNK
ROWS_PER_TILE = N_PAD // 16


def _alphas(h, avs_ref, avd_ref, asrc_ref, adst_ref, bigm_ref):
    asrc = jnp.sum(h * avs_ref[...][None, :], axis=1)
    adst = jnp.sum(h * avd_ref[...][None, :], axis=1)
    asrc_ref[...] = asrc
    adst_ref[...] = adst
    msum = jnp.max(asrc) + jnp.max(adst)
    bigm_ref[...] = jnp.full((16,), jnp.maximum(msum, 0.2 * msum))


def _pre_body(x_ref, w_ref, avs_ref, avd_ref,
              h_ref, asrc_ref, adst_ref, bigm_ref):
    h = jnp.dot(x_ref[...], w_ref[...], preferred_element_type=jnp.float32)
    h_ref[...] = h
    _alphas(h, avs_ref, avd_ref, asrc_ref, adst_ref, bigm_ref)


def _tc_pre(x, w, avs, avd):
    return pl.pallas_call(
        _pre_body,
        out_shape=[
            jax.ShapeDtypeStruct((N_PAD, D), jnp.float32),
            jax.ShapeDtypeStruct((N_PAD,), jnp.float32),
            jax.ShapeDtypeStruct((N_PAD,), jnp.float32),
            jax.ShapeDtypeStruct((16,), jnp.float32),
        ],
    )(x, w, avs, avd)


def _denb_body(partd_ref, out_ref):
    pd = partd_ref[0, 0, 0] + partd_ref[1, 0, 0]
    out_ref[...] = jnp.broadcast_to(pd, (128, 128)).T


def _den_bcast(partd):
    nb = N_PAD // 128
    return pl.pallas_call(
        _denb_body,
        grid=(nb,),
        in_specs=[pl.BlockSpec((2, 1, 1, 128), lambda i: (0, i, 0, 0))],
        out_specs=pl.BlockSpec((128, 128), lambda i: (i, 0)),
        out_shape=jax.ShapeDtypeStruct((N_PAD, 128), jnp.float32),
    )(partd.reshape(2, nb, 1, 128))


def _merge(part_ref, denb_ref, b_ref):
    num = part_ref[0] + part_ref[1]
    den = denb_ref[...]
    return jnp.maximum(num / (den + 1e-16) + b_ref[...][None, :], 0.0)


def _mid_body(part_ref, partd_ref, b_ref, w_ref, avs_ref, avd_ref,
              h_ref, asrc_ref, adst_ref, bigm_ref):
    g = _merge(part_ref, partd_ref, b_ref)
    h = jnp.dot(g, w_ref[...], preferred_element_type=jnp.float32)
    h_ref[...] = h
    _alphas(h, avs_ref, avd_ref, asrc_ref, adst_ref, bigm_ref)


def _tc_mid(part, partd, b, w, avs, avd):
    return pl.pallas_call(
        _mid_body,
        out_shape=[
            jax.ShapeDtypeStruct((N_PAD, D), jnp.float32),
            jax.ShapeDtypeStruct((N_PAD,), jnp.float32),
            jax.ShapeDtypeStruct((N_PAD,), jnp.float32),
            jax.ShapeDtypeStruct((16,), jnp.float32),
        ],
    )(part, partd, b, w, avs, avd)


def _post_body(part_ref, partd_ref, b_ref, wc_ref, bc_ref, out_ref):
    g = _merge(part_ref, partd_ref, b_ref)
    logits = jnp.dot(g, wc_ref[...], preferred_element_type=jnp.float32)
    logits = logits + bc_ref[...][None, :]
    m = jnp.max(logits, axis=1, keepdims=True)
    lse = jnp.log(jnp.sum(jnp.exp(logits - m), axis=1, keepdims=True)) + m
    out_ref[...] = logits - lse


def _tc_post(part, partd, b, wc, bc):
    nc = wc.shape[1]
    return pl.pallas_call(
        _post_body,
        out_shape=jax.ShapeDtypeStruct((N_PAD, nc), jnp.float32),
    )(part, partd, b, wc, bc)


N_DROW = N_PAD // 128


def _sc_body(h_hbm, asrc_hbm, adst_hbm, bigm_hbm, src_hbm, dst_hbm,
             out_hbm, outd_hbm,
             asrc_v, adst_v, bigm_v, src_v, dst_v, rows_v, den_v, idx80_v,
             acc_sh, accd_sh, gsem, ssem):
    c = lax.axis_index("c")
    s = lax.axis_index("s")
    wid = c * 16 + s
    base_e = wid * CHUNK

    pltpu.sync_copy(asrc_hbm, asrc_v)
    pltpu.sync_copy(adst_hbm, adst_v)
    pltpu.sync_copy(bigm_hbm, bigm_v)

    big_m = bigm_v[...]

    zf = jnp.zeros((16,), jnp.float32)
    iota16 = lax.iota(jnp.int32, 16)

    def _zden(i, carry):
        for j in range(D // 16):
            den_v[i, pl.ds(j * 16, 16)] = zf
        return carry

    lax.fori_loop(0, N_DROW, _zden, 0)

    def _zidx(i, carry):
        idx80_v[pl.ds(i * 16, 16)] = iota16 + i * 16
        return carry

    lax.fori_loop(0, N_DROW // 16, _zidx, 0)

    zsrc = den_v.at[pl.ds(0, 16)]

    def _zcp(g, carry):
        pltpu.sync_copy(zsrc, acc_sh.at[pl.ds(s * ROWS_PER_TILE + g * 16, 16)])
        return carry

    lax.fori_loop(0, ROWS_PER_TILE // 16, _zcp, 0)

    @pl.when(s < N_DROW // 16)
    def _():
        pltpu.sync_copy(zsrc, accd_sh.at[pl.ds(s * 16, 16)])

    plsc.subcore_barrier()

    zeros16i = jnp.zeros((16,), jnp.int32)
    n_blk = STAGE // G

    def _gissue(stage_blk, par):
        idx = src_v.at[pl.ds(stage_blk * G, G)]
        pltpu.async_copy(h_hbm.at[idx], rows_v.at[pl.ds(par * G, G)], gsem)

    def _gwait():
        pltpu.make_async_copy(h_hbm.at[src_v.at[pl.ds(0, G)]],
                              rows_v.at[pl.ds(0, G)], gsem).wait()

    def _sdrain():
        for _ in range(G // 16):
            pltpu.make_async_copy(rows_v.at[pl.ds(0, 16)],
                                  acc_sh.at[zeros16i], ssem).wait()

    def _process(g, par):
        @pl.when(g < n_blk - 1)
        def _():
            _gissue(g + 1, 1 - par)

        for sub in range(G // 16):
            sl = pl.ds(g * G + sub * 16, 16)
            s16 = src_v[sl]
            d16 = dst_v[sl]
            av = plsc.load_gather(asrc_v, [s16])
            bv = plsc.load_gather(adst_v, [d16])
            e = av + bv
            e = jnp.where(e >= 0.0, e, 0.2 * e) - big_m
            p = jnp.exp(e)
            dmod = jnp.bitwise_and(d16, 127)
            drow = lax.shift_right_logical(d16, 7)
            plsc.addupdate_scatter(den_v, [drow, dmod], p)
            rbase = par * G + sub * 16
            for r in range(16):
                rsel = jnp.full((16,), r, jnp.int32)
                prv = p.at[rsel].get(mode="promise_in_bounds")
                for j in range(D // 16):
                    cs = pl.ds(j * 16, 16)
                    rows_v[rbase + r, cs] = rows_v[rbase + r, cs] * prv
            pltpu.async_copy(rows_v.at[pl.ds(rbase, 16)],
                             acc_sh.at[d16], ssem, add=True)

    def _blk(g, carry):
        par = jnp.bitwise_and(g, 1)

        @pl.when(g > 0)
        def _():
            _sdrain()

        _gwait()

        @pl.when(par == 0)
        def _():
            _process(g, 0)

        @pl.when(par == 1)
        def _():
            _process(g, 1)

        return carry

    def _stage(hh, carry):
        pltpu.sync_copy(src_hbm.at[pl.ds(base_e + hh * STAGE, STAGE)], src_v)
        pltpu.sync_copy(dst_hbm.at[pl.ds(base_e + hh * STAGE, STAGE)], dst_v)
        _gissue(0, 0)
        lax.fori_loop(0, n_blk, _blk, 0)
        _sdrain()
        return carry

    lax.fori_loop(0, CHUNK // STAGE, _stage, 0)

    pltpu.sync_copy(den_v, accd_sh.at[idx80_v], add=True)
    plsc.subcore_barrier()

    pltpu.sync_copy(acc_sh.at[pl.ds(s * ROWS_PER_TILE, ROWS_PER_TILE)],
                    out_hbm.at[c].at[pl.ds(s * ROWS_PER_TILE, ROWS_PER_TILE)])

    @pl.when(s < N_DROW // 16)
    def _():
        pltpu.sync_copy(accd_sh.at[pl.ds(s * 16, 16)],
                        outd_hbm.at[c].at[pl.ds(s * 16, 16)])


_sc_edge = pl.kernel(
    _sc_body,
    out_type=[
        jax.ShapeDtypeStruct((2, N_PAD, D), jnp.float32),
        jax.ShapeDtypeStruct((2, N_DROW, 128), jnp.float32),
    ],
    mesh=plsc.VectorSubcoreMesh(core_axis_name="c", subcore_axis_name="s"),
    scratch_types=[
        pltpu.VMEM((N_PAD,), jnp.float32),
        pltpu.VMEM((N_PAD,), jnp.float32),
        pltpu.VMEM((16,), jnp.float32),
        pltpu.VMEM((STAGE,), jnp.int32),
        pltpu.VMEM((STAGE,), jnp.int32),
        pltpu.VMEM((2 * G, D), jnp.float32),
        pltpu.VMEM((N_DROW, D), jnp.float32),
        pltpu.VMEM((N_DROW,), jnp.int32),
        pltpu.VMEM_SHARED((N_PAD, D), jnp.float32),
        pltpu.VMEM_SHARED((N_DROW, 128), jnp.float32),
        pltpu.SemaphoreType.DMA,
        pltpu.SemaphoreType.DMA,
    ],
    compiler_params=pltpu.CompilerParams(needs_layout_passes=False),
)


@jax.jit
def _run(x, edge_index, W0, a0s, a0d, b0, W1, a1s, a1d, b1, Wc, bc):
    n = x.shape[0]
    loops = jnp.arange(n, dtype=jnp.int32)
    src = jnp.concatenate([
        edge_index[0].astype(jnp.int32), loops,
        jnp.zeros((E_PAD - E_TOT,), jnp.int32)])
    dst = jnp.concatenate([
        edge_index[1].astype(jnp.int32), loops,
        jnp.full((E_PAD - E_TOT,), n, jnp.int32)])
    x_pad = jnp.zeros((N_PAD, D), jnp.float32).at[:n].set(x)

    h0, asrc0, adst0, bigm0 = _tc_pre(x_pad, W0, a0s, a0d)
    part0, partd0 = _sc_edge(h0, asrc0, adst0, bigm0, src, dst)
    h1, asrc1, adst1, bigm1 = _tc_mid(part0, _den_bcast(partd0), b0, W1, a1s, a1d)
    part1, partd1 = _sc_edge(h1, asrc1, adst1, bigm1, src, dst)
    out = _tc_post(part1, _den_bcast(partd1), b1, Wc, bc)
    return out[:n]


def kernel(x, edge_index, W0, a0s, a0d, b0, W1, a1s, a1d, b1, Wc, bc):
    return _run(x, edge_index, W0, a0s, a0d, b0, W1, a1s, a1d, b1, Wc, bc)


# speedup vs baseline: 1.2136x; 1.2136x over previous
import functools
import jax
import jax.numpy as jnp
from jax import lax
from jax.experimental import pallas as pl
from jax.experimental.pallas import tpu as pltpu
from jax.experimental.pallas import tpu_sc as plsc

N_NODES = 10000
D = 128
N_PAD = 10240
ACC_W = 144
E_RAW = 320000
E_TOT = E_RAW + N_NODES
N_WORKERS = 32
CHUNK = 10368
STAGE = 1296
G = 48
E_PAD = N_WORKERS * CHUNK
ROWS_PER_TILE = N_PAD // 16


def _alphas(h, avs_ref, avd_ref, asrc_ref, adst_ref, bigm_ref):
    asrc = jnp.sum(h * avs_ref[...][None, :], axis=1)
    adst = jnp.sum(h * avd_ref[...][None, :], axis=1)
    asrc_ref[...] = asrc
    adst_ref[...] = adst
    msum = jnp.max(asrc) + jnp.max(adst)
    bigm_ref[...] = jnp.full((16,), jnp.maximum(msum, 0.2 * msum))


def _pre_body(x_ref, w_ref, avs_ref, avd_ref,
              h_ref, asrc_ref, adst_ref, bigm_ref):
    h = jnp.dot(x_ref[...], w_ref[...], preferred_element_type=jnp.float32)
    h_ref[...] = h
    _alphas(h, avs_ref, avd_ref, asrc_ref, adst_ref, bigm_ref)


def _tc_pre(x, w, avs, avd):
    return pl.pallas_call(
        _pre_body,
        out_shape=[
            jax.ShapeDtypeStruct((N_PAD, D), jnp.float32),
            jax.ShapeDtypeStruct((N_PAD,), jnp.float32),
            jax.ShapeDtypeStruct((N_PAD,), jnp.float32),
            jax.ShapeDtypeStruct((16,), jnp.float32),
        ],
    )(x, w, avs, avd)


def _denb_body(partd_ref, out_ref):
    pd = partd_ref[0, 0, 0] + partd_ref[1, 0, 0]
    out_ref[...] = jnp.broadcast_to(pd, (128, 128)).T


def _den_bcast(partd):
    nb = N_PAD // 128
    return pl.pallas_call(
        _denb_body,
        grid=(nb,),
        in_specs=[pl.BlockSpec((2, 1, 1, 128), lambda i: (0, i, 0, 0))],
        out_specs=pl.BlockSpec((128, 128), lambda i: (i, 0)),
        out_shape=jax.ShapeDtypeStruct((N_PAD, 128), jnp.float32),
    )(partd.reshape(2, nb, 1, 128))


def _merge(part_ref, denb_ref, b_ref):
    num = part_ref[0] + part_ref[1]
    den = denb_ref[...]
    return jnp.maximum(num / (den + 1e-16) + b_ref[...][None, :], 0.0)


def _mid_body(part_ref, partd_ref, b_ref, w_ref, avs_ref, avd_ref,
              h_ref, asrc_ref, adst_ref, bigm_ref):
    g = _merge(part_ref, partd_ref, b_ref)
    h = jnp.dot(g, w_ref[...], preferred_element_type=jnp.float32)
    h_ref[...] = h
    _alphas(h, avs_ref, avd_ref, asrc_ref, adst_ref, bigm_ref)


def _tc_mid(part, partd, b, w, avs, avd):
    return pl.pallas_call(
        _mid_body,
        out_shape=[
            jax.ShapeDtypeStruct((N_PAD, D), jnp.float32),
            jax.ShapeDtypeStruct((N_PAD,), jnp.float32),
            jax.ShapeDtypeStruct((N_PAD,), jnp.float32),
            jax.ShapeDtypeStruct((16,), jnp.float32),
        ],
    )(part, partd, b, w, avs, avd)


def _post_body(part_ref, partd_ref, b_ref, wc_ref, bc_ref, out_ref):
    g = _merge(part_ref, partd_ref, b_ref)
    logits = jnp.dot(g, wc_ref[...], preferred_element_type=jnp.float32)
    logits = logits + bc_ref[...][None, :]
    m = jnp.max(logits, axis=1, keepdims=True)
    lse = jnp.log(jnp.sum(jnp.exp(logits - m), axis=1, keepdims=True)) + m
    out_ref[...] = logits - lse


def _tc_post(part, partd, b, wc, bc):
    nc = wc.shape[1]
    return pl.pallas_call(
        _post_body,
        out_shape=jax.ShapeDtypeStruct((N_PAD, nc), jnp.float32),
    )(part, partd, b, wc, bc)


N_DROW = N_PAD // 128


def _sc_body(h_hbm, asrc_hbm, adst_hbm, bigm_hbm, src_hbm, dst_hbm,
             out_hbm, outd_hbm,
             asrc_v, adst_v, bigm_v, src_v, dst_v, rows_v, den_v, idx80_v,
             acc_sh, accd_sh, gsem, ssem):
    c = lax.axis_index("c")
    s = lax.axis_index("s")
    wid = c * 16 + s
    base_e = wid * CHUNK

    pltpu.sync_copy(asrc_hbm, asrc_v)
    pltpu.sync_copy(adst_hbm, adst_v)
    pltpu.sync_copy(bigm_hbm, bigm_v)

    big_m = bigm_v[...]

    zf = jnp.zeros((16,), jnp.float32)
    iota16 = lax.iota(jnp.int32, 16)

    def _zden(i, carry):
        for j in range(D // 16):
            den_v[i, pl.ds(j * 16, 16)] = zf
        return carry

    lax.fori_loop(0, N_DROW, _zden, 0)

    def _zidx(i, carry):
        idx80_v[pl.ds(i * 16, 16)] = iota16 + i * 16
        return carry

    lax.fori_loop(0, N_DROW // 16, _zidx, 0)

    zsrc = den_v.at[pl.ds(0, 16)]

    def _zcp(g, carry):
        pltpu.sync_copy(zsrc, acc_sh.at[pl.ds(s * ROWS_PER_TILE + g * 16, 16)])
        return carry

    lax.fori_loop(0, ROWS_PER_TILE // 16, _zcp, 0)

    @pl.when(s < N_DROW // 16)
    def _():
        pltpu.sync_copy(zsrc, accd_sh.at[pl.ds(s * 16, 16)])

    plsc.subcore_barrier()

    zeros16i = jnp.zeros((16,), jnp.int32)
    n_blk = STAGE // G

    def _gissue(stage_blk, par):
        idx = src_v.at[pl.ds(stage_blk * G, G)]
        pltpu.async_copy(h_hbm.at[idx], rows_v.at[pl.ds(par * G, G)], gsem)

    def _gwait():
        pltpu.make_async_copy(h_hbm.at[src_v.at[pl.ds(0, G)]],
                              rows_v.at[pl.ds(0, G)], gsem).wait()

    def _sdrain():
        for _ in range(G // 16):
            pltpu.make_async_copy(rows_v.at[pl.ds(0, 16)],
                                  acc_sh.at[zeros16i], ssem).wait()

    def _blk(g, carry):
        par = jnp.bitwise_and(g, 1)

        @pl.when(g > 0)
        def _():
            _sdrain()

        _gwait()

        @pl.when(g < n_blk - 1)
        def _():
            _gissue(g + 1, 1 - par)

        for sub in range(G // 16):
            sl = pl.ds(g * G + sub * 16, 16)
            s16 = src_v[sl]
            d16 = dst_v[sl]
            av = plsc.load_gather(asrc_v, [s16])
            bv = plsc.load_gather(adst_v, [d16])
            e = av + bv
            e = jnp.where(e >= 0.0, e, 0.2 * e) - big_m
            p = jnp.exp(e)
            dmod = jnp.bitwise_and(d16, 127)
            drow = lax.shift_right_logical(d16, 7)
            plsc.addupdate_scatter(den_v, [drow, dmod], p)
            rbase = par * G + sub * 16
            for r in range(16):
                rsel = jnp.full((16,), r, jnp.int32)
                prv = p.at[rsel].get(mode="promise_in_bounds")
                for j in range(D // 16):
                    cs = pl.ds(j * 16, 16)
                    rows_v[rbase + r, cs] = rows_v[rbase + r, cs] * prv
            pltpu.async_copy(rows_v.at[pl.ds(rbase, 16)],
                             acc_sh.at[d16], ssem, add=True)
        return carry

    def _stage(hh, carry):
        pltpu.sync_copy(src_hbm.at[pl.ds(base_e + hh * STAGE, STAGE)], src_v)
        pltpu.sync_copy(dst_hbm.at[pl.ds(base_e + hh * STAGE, STAGE)], dst_v)
        _gissue(0, 0)
        lax.fori_loop(0, n_blk, _blk, 0)
        _sdrain()
        return carry

    lax.fori_loop(0, CHUNK // STAGE, _stage, 0)

    pltpu.sync_copy(den_v, accd_sh.at[idx80_v], add=True)
    plsc.subcore_barrier()

    pltpu.sync_copy(acc_sh.at[pl.ds(s * ROWS_PER_TILE, ROWS_PER_TILE)],
                    out_hbm.at[c].at[pl.ds(s * ROWS_PER_TILE, ROWS_PER_TILE)])

    @pl.when(s < N_DROW // 16)
    def _():
        pltpu.sync_copy(accd_sh.at[pl.ds(s * 16, 16)],
                        outd_hbm.at[c].at[pl.ds(s * 16, 16)])


_sc_edge = pl.kernel(
    _sc_body,
    out_type=[
        jax.ShapeDtypeStruct((2, N_PAD, D), jnp.float32),
        jax.ShapeDtypeStruct((2, N_DROW, 128), jnp.float32),
    ],
    mesh=plsc.VectorSubcoreMesh(core_axis_name="c", subcore_axis_name="s"),
    scratch_types=[
        pltpu.VMEM((N_PAD,), jnp.float32),
        pltpu.VMEM((N_PAD,), jnp.float32),
        pltpu.VMEM((16,), jnp.float32),
        pltpu.VMEM((STAGE,), jnp.int32),
        pltpu.VMEM((STAGE,), jnp.int32),
        pltpu.VMEM((2 * G, D), jnp.float32),
        pltpu.VMEM((N_DROW, D), jnp.float32),
        pltpu.VMEM((N_DROW,), jnp.int32),
        pltpu.VMEM_SHARED((N_PAD, D), jnp.float32),
        pltpu.VMEM_SHARED((N_DROW, 128), jnp.float32),
        pltpu.SemaphoreType.DMA,
        pltpu.SemaphoreType.DMA,
    ],
    compiler_params=pltpu.CompilerParams(needs_layout_passes=False),
)


@jax.jit
def _run(x, edge_index, W0, a0s, a0d, b0, W1, a1s, a1d, b1, Wc, bc):
    n = x.shape[0]
    loops = jnp.arange(n, dtype=jnp.int32)
    src = jnp.concatenate([
        edge_index[0].astype(jnp.int32), loops,
        jnp.zeros((E_PAD - E_TOT,), jnp.int32)])
    dst = jnp.concatenate([
        edge_index[1].astype(jnp.int32), loops,
        jnp.full((E_PAD - E_TOT,), n, jnp.int32)])
    x_pad = jnp.zeros((N_PAD, D), jnp.float32).at[:n].set(x)

    h0, asrc0, adst0, bigm0 = _tc_pre(x_pad, W0, a0s, a0d)
    part0, partd0 = _sc_edge(h0, asrc0, adst0, bigm0, src, dst)
    h1, asrc1, adst1, bigm1 = _tc_mid(part0, _den_bcast(partd0), b0, W1, a1s, a1d)
    part1, partd1 = _sc_edge(h1, asrc1, adst1, bigm1, src, dst)
    out = _tc_post(part1, _den_bcast(partd1), b1, Wc, bc)
    return out[:n]


def kernel(x, edge_index, W0, a0s, a0d, b0, W1, a1s, a1d, b1, Wc, bc):
    return _run(x, edge_index, W0, a0s, a0d, b0, W1, a1s, a1d, b1, Wc, bc)


# speedup vs baseline: 1.4353x; 1.1827x over previous
import functools
import jax
import jax.numpy as jnp
from jax import lax
from jax.experimental import pallas as pl
from jax.experimental.pallas import tpu as pltpu
from jax.experimental.pallas import tpu_sc as plsc

N_NODES = 10000
D = 128
N_PAD = 10240
ACC_W = 144
E_RAW = 320000
E_TOT = E_RAW + N_NODES
N_WORKERS = 32
CHUNK = 10368
STAGE = 1728
G = 32
NBUF = 3
E_PAD = N_WORKERS * CHUNK
ROWS_PER_TILE = N_PAD // 16


def _alphas(h, avs_ref, avd_ref, asrc_ref, adst_ref, bigm_ref):
    asrc = jnp.sum(h * avs_ref[...][None, :], axis=1)
    adst = jnp.sum(h * avd_ref[...][None, :], axis=1)
    asrc_ref[...] = asrc
    adst_ref[...] = adst
    msum = jnp.max(asrc) + jnp.max(adst)
    bigm_ref[...] = jnp.full((16,), jnp.maximum(msum, 0.2 * msum))


def _pre_body(x_ref, w_ref, avs_ref, avd_ref,
              h_ref, asrc_ref, adst_ref, bigm_ref):
    h = jnp.dot(x_ref[...], w_ref[...], preferred_element_type=jnp.float32)
    h_ref[...] = h
    _alphas(h, avs_ref, avd_ref, asrc_ref, adst_ref, bigm_ref)


def _tc_pre(x, w, avs, avd):
    return pl.pallas_call(
        _pre_body,
        out_shape=[
            jax.ShapeDtypeStruct((N_PAD, D), jnp.float32),
            jax.ShapeDtypeStruct((N_PAD,), jnp.float32),
            jax.ShapeDtypeStruct((N_PAD,), jnp.float32),
            jax.ShapeDtypeStruct((16,), jnp.float32),
        ],
    )(x, w, avs, avd)


def _denb_body(partd_ref, out_ref):
    pd = partd_ref[0, 0, 0] + partd_ref[1, 0, 0]
    out_ref[...] = jnp.broadcast_to(pd, (128, 128)).T


def _den_bcast(partd):
    nb = N_PAD // 128
    return pl.pallas_call(
        _denb_body,
        grid=(nb,),
        in_specs=[pl.BlockSpec((2, 1, 1, 128), lambda i: (0, i, 0, 0))],
        out_specs=pl.BlockSpec((128, 128), lambda i: (i, 0)),
        out_shape=jax.ShapeDtypeStruct((N_PAD, 128), jnp.float32),
    )(partd.reshape(2, nb, 1, 128))


def _merge(part_ref, denb_ref, b_ref):
    num = part_ref[0] + part_ref[1]
    den = denb_ref[...]
    return jnp.maximum(num / (den + 1e-16) + b_ref[...][None, :], 0.0)


def _mid_body(part_ref, partd_ref, b_ref, w_ref, avs_ref, avd_ref,
              h_ref, asrc_ref, adst_ref, bigm_ref):
    g = _merge(part_ref, partd_ref, b_ref)
    h = jnp.dot(g, w_ref[...], preferred_element_type=jnp.float32)
    h_ref[...] = h
    _alphas(h, avs_ref, avd_ref, asrc_ref, adst_ref, bigm_ref)


def _tc_mid(part, partd, b, w, avs, avd):
    return pl.pallas_call(
        _mid_body,
        out_shape=[
            jax.ShapeDtypeStruct((N_PAD, D), jnp.float32),
            jax.ShapeDtypeStruct((N_PAD,), jnp.float32),
            jax.ShapeDtypeStruct((N_PAD,), jnp.float32),
            jax.ShapeDtypeStruct((16,), jnp.float32),
        ],
    )(part, partd, b, w, avs, avd)


def _post_body(part_ref, partd_ref, b_ref, wc_ref, bc_ref, out_ref):
    g = _merge(part_ref, partd_ref, b_ref)
    logits = jnp.dot(g, wc_ref[...], preferred_element_type=jnp.float32)
    logits = logits + bc_ref[...][None, :]
    m = jnp.max(logits, axis=1, keepdims=True)
    lse = jnp.log(jnp.sum(jnp.exp(logits - m), axis=1, keepdims=True)) + m
    out_ref[...] = logits - lse


def _tc_post(part, partd, b, wc, bc):
    nc = wc.shape[1]
    return pl.pallas_call(
        _post_body,
        out_shape=jax.ShapeDtypeStruct((N_PAD, nc), jnp.float32),
    )(part, partd, b, wc, bc)


N_DROW = N_PAD // 128


def _sc_body(h_hbm, asrc_hbm, adst_hbm, bigm_hbm, src_hbm, dst_hbm,
             out_hbm, outd_hbm,
             asrc_v, adst_v, bigm_v, src_v, dst_v, rows_v, den_v, idx80_v,
             acc_sh, accd_sh, gsem_a, gsem_b, ssem):
    c = lax.axis_index("c")
    s = lax.axis_index("s")
    wid = c * 16 + s
    base_e = wid * CHUNK

    pltpu.sync_copy(asrc_hbm, asrc_v)
    pltpu.sync_copy(adst_hbm, adst_v)
    pltpu.sync_copy(bigm_hbm, bigm_v)

    big_m = bigm_v[...]

    zf = jnp.zeros((16,), jnp.float32)
    iota16 = lax.iota(jnp.int32, 16)

    def _zden(i, carry):
        for j in range(D // 16):
            den_v[i, pl.ds(j * 16, 16)] = zf
        return carry

    lax.fori_loop(0, N_DROW, _zden, 0)

    def _zidx(i, carry):
        idx80_v[pl.ds(i * 16, 16)] = iota16 + i * 16
        return carry

    lax.fori_loop(0, N_DROW // 16, _zidx, 0)

    zsrc = den_v.at[pl.ds(0, 16)]

    def _zcp(g, carry):
        pltpu.sync_copy(zsrc, acc_sh.at[pl.ds(s * ROWS_PER_TILE + g * 16, 16)])
        return carry

    lax.fori_loop(0, ROWS_PER_TILE // 16, _zcp, 0)

    @pl.when(s < N_DROW // 16)
    def _():
        pltpu.sync_copy(zsrc, accd_sh.at[pl.ds(s * 16, 16)])

    plsc.subcore_barrier()

    zeros16i = jnp.zeros((16,), jnp.int32)
    n_blk = STAGE // G

    def _gissue(stage_blk, buf, sem):
        idx = src_v.at[pl.ds(stage_blk * G, G)]
        pltpu.async_copy(h_hbm.at[idx], rows_v.at[pl.ds(buf * G, G)], sem)

    def _gwait(sem):
        pltpu.make_async_copy(h_hbm.at[src_v.at[pl.ds(0, G)]],
                              rows_v.at[pl.ds(0, G)], sem).wait()

    def _sdrain():
        for _ in range(G // 16):
            pltpu.make_async_copy(rows_v.at[pl.ds(0, 16)],
                                  acc_sh.at[zeros16i], ssem).wait()

    def _blk(g, carry):
        par = jnp.bitwise_and(g, 1)
        buf = lax.rem(g, NBUF)
        nbuf = lax.rem(g + 2, NBUF)

        @pl.when(g > 0)
        def _():
            _sdrain()

        @pl.when(par == 0)
        def _():
            _gwait(gsem_a)

            @pl.when(g < n_blk - 2)
            def _():
                _gissue(g + 2, nbuf, gsem_a)

        @pl.when(par == 1)
        def _():
            _gwait(gsem_b)

            @pl.when(g < n_blk - 2)
            def _():
                _gissue(g + 2, nbuf, gsem_b)

        for sub in range(G // 16):
            sl = pl.ds(g * G + sub * 16, 16)
            s16 = src_v[sl]
            d16 = dst_v[sl]
            av = plsc.load_gather(asrc_v, [s16])
            bv = plsc.load_gather(adst_v, [d16])
            e = av + bv
            e = jnp.where(e >= 0.0, e, 0.2 * e) - big_m
            p = jnp.exp(e)
            dmod = jnp.bitwise_and(d16, 127)
            drow = lax.shift_right_logical(d16, 7)
            plsc.addupdate_scatter(den_v, [drow, dmod], p)
            rbase = buf * G + sub * 16
            for r in range(16):
                rsel = jnp.full((16,), r, jnp.int32)
                prv = p.at[rsel].get(mode="promise_in_bounds")
                for j in range(D // 16):
                    cs = pl.ds(j * 16, 16)
                    rows_v[rbase + r, cs] = rows_v[rbase + r, cs] * prv
            pltpu.async_copy(rows_v.at[pl.ds(rbase, 16)],
                             acc_sh.at[d16], ssem, add=True)
        return carry

    def _stage(hh, carry):
        pltpu.sync_copy(src_hbm.at[pl.ds(base_e + hh * STAGE, STAGE)], src_v)
        pltpu.sync_copy(dst_hbm.at[pl.ds(base_e + hh * STAGE, STAGE)], dst_v)
        _gissue(0, 0, gsem_a)
        _gissue(1, 1, gsem_b)
        lax.fori_loop(0, n_blk, _blk, 0)
        _sdrain()
        return carry

    lax.fori_loop(0, CHUNK // STAGE, _stage, 0)

    pltpu.sync_copy(den_v, accd_sh.at[idx80_v], add=True)
    plsc.subcore_barrier()

    pltpu.sync_copy(acc_sh.at[pl.ds(s * ROWS_PER_TILE, ROWS_PER_TILE)],
                    out_hbm.at[c].at[pl.ds(s * ROWS_PER_TILE, ROWS_PER_TILE)])

    @pl.when(s < N_DROW // 16)
    def _():
        pltpu.sync_copy(accd_sh.at[pl.ds(s * 16, 16)],
                        outd_hbm.at[c].at[pl.ds(s * 16, 16)])


_sc_edge = pl.kernel(
    _sc_body,
    out_type=[
        jax.ShapeDtypeStruct((2, N_PAD, D), jnp.float32),
        jax.ShapeDtypeStruct((2, N_DROW, 128), jnp.float32),
    ],
    mesh=plsc.VectorSubcoreMesh(core_axis_name="c", subcore_axis_name="s"),
    scratch_types=[
        pltpu.VMEM((N_PAD,), jnp.float32),
        pltpu.VMEM((N_PAD,), jnp.float32),
        pltpu.VMEM((16,), jnp.float32),
        pltpu.VMEM((STAGE,), jnp.int32),
        pltpu.VMEM((STAGE,), jnp.int32),
        pltpu.VMEM((NBUF * G, D), jnp.float32),
        pltpu.VMEM((N_DROW, D), jnp.float32),
        pltpu.VMEM((N_DROW,), jnp.int32),
        pltpu.VMEM_SHARED((N_PAD, D), jnp.float32),
        pltpu.VMEM_SHARED((N_DROW, 128), jnp.float32),
        pltpu.SemaphoreType.DMA,
        pltpu.SemaphoreType.DMA,
        pltpu.SemaphoreType.DMA,
    ],
    compiler_params=pltpu.CompilerParams(needs_layout_passes=False),
)


@jax.jit
def _run(x, edge_index, W0, a0s, a0d, b0, W1, a1s, a1d, b1, Wc, bc):
    n = x.shape[0]
    loops = jnp.arange(n, dtype=jnp.int32)
    src = jnp.concatenate([
        edge_index[0].astype(jnp.int32), loops,
        jnp.zeros((E_PAD - E_TOT,), jnp.int32)])
    dst = jnp.concatenate([
        edge_index[1].astype(jnp.int32), loops,
        jnp.full((E_PAD - E_TOT,), n, jnp.int32)])
    x_pad = jnp.zeros((N_PAD, D), jnp.float32).at[:n].set(x)

    h0, asrc0, adst0, bigm0 = _tc_pre(x_pad, W0, a0s, a0d)
    part0, partd0 = _sc_edge(h0, asrc0, adst0, bigm0, src, dst)
    h1, asrc1, adst1, bigm1 = _tc_mid(part0, _den_bcast(partd0), b0, W1, a1s, a1d)
    part1, partd1 = _sc_edge(h1, asrc1, adst1, bigm1, src, dst)
    out = _tc_post(part1, _den_bcast(partd1), b1, Wc, bc)
    return out[:n]


def kernel(x, edge_index, W0, a0s, a0d, b0, W1, a1s, a1d, b1, Wc, bc):
    return _run(x, edge_index, W0, a0s, a0d, b0, W1, a1s, a1d, b1, Wc, bc)
